# Initial kernel scaffold; baseline (speedup 1.0000x reference)
import jax
import jax.numpy as jnp
from jax.experimental import pallas as pl


def kernel(x, edge_index, edge_attr, type_embed, feat_W, feat_b, edge_W, edge_b, W1_0, b1_0, W2_0, b2_0, eps_0, W1_1, b1_1, W2_1, b2_1, eps_1, W1_2, b1_2, W2_2, b2_2, eps_2):
    raise NotImplementedError("write your pallas kernel here")


# speedup vs baseline: 3.8818x; 3.8818x over previous
import functools

import jax
import jax.numpy as jnp
from jax import lax
from jax.experimental import pallas as pl
from jax.experimental.pallas import tpu as pltpu
from jax.experimental.pallas import tpu_sc as plsc

N = 100000
E = 1600000
NT = 32
NF = 9
NEF = 16
H = 32
PE = 16

ROWS = E // 128
HALF = N // 2
DUMMY = HALF
SP_ROWS = 50048
STRIPE = SP_ROWS // 16
CHUNK_ROWS = 8
SUB_ROWS = 2
SUB_E = SUB_ROWS * 128
NCHUNKS = ROWS // CHUNK_ROWS
TAIL_ROWS = ROWS - NCHUNKS * CHUNK_ROWS
ZCOPIES = STRIPE // SUB_E
ZREM = STRIPE - ZCOPIES * SUB_E


def _node_embed_body(x_ref, te_ref, w_ref, b_ref, o_ref):
    xb = x_ref[...]
    t = xb[:, 0:1]
    iot = lax.broadcasted_iota(jnp.int32, (1, NT), 1).astype(jnp.float32)
    oh = (t == iot).astype(jnp.float32)
    h = jnp.dot(oh, te_ref[...], preferred_element_type=jnp.float32)
    h = h + jnp.dot(xb, w_ref[...], preferred_element_type=jnp.float32)
    o_ref[...] = h + b_ref[...]


def _edge_embed_body(ea_ref, w_ref, b_ref, o_ref):
    o_ref[...] = (
        jnp.dot(ea_ref[...], w_ref[...], preferred_element_type=jnp.float32)
        + b_ref[...]
    )


def _mlp_body(h_ref, agg_ref, w1_ref, b1_ref, w2_ref, b2_ref, eps_ref, o_ref,
              *, relu_out):
    z = (1.0 + eps_ref[0, 0]) * h_ref[...] + agg_ref[...]
    z = jnp.maximum(
        jnp.dot(z, w1_ref[...], preferred_element_type=jnp.float32) + b1_ref[...],
        0.0,
    )
    out = jnp.dot(z, w2_ref[...], preferred_element_type=jnp.float32) + b2_ref[...]
    if relu_out:
        out = jnp.maximum(out, 0.0)
    o_ref[...] = out


BN = 2000
BE = 8000


def _node_embed(x, type_embed, w_pad, b):
    return pl.pallas_call(
        _node_embed_body,
        grid=(N // BN,),
        in_specs=[
            pl.BlockSpec((BN, NF), lambda i: (i, 0)),
            pl.BlockSpec((NT, H), lambda i: (0, 0)),
            pl.BlockSpec((NF, H), lambda i: (0, 0)),
            pl.BlockSpec((1, H), lambda i: (0, 0)),
        ],
        out_specs=pl.BlockSpec((BN, H), lambda i: (i, 0)),
        out_shape=jax.ShapeDtypeStruct((N, H), jnp.float32),
    )(x, type_embed, w_pad, b)


def _edge_embed(edge_attr, w, b):
    return pl.pallas_call(
        _edge_embed_body,
        grid=(E // BE,),
        in_specs=[
            pl.BlockSpec((BE, NEF), lambda i: (i, 0)),
            pl.BlockSpec((NEF, H), lambda i: (0, 0)),
            pl.BlockSpec((1, H), lambda i: (0, 0)),
        ],
        out_specs=pl.BlockSpec((BE, H), lambda i: (i, 0)),
        out_shape=jax.ShapeDtypeStruct((E, H), jnp.float32),
    )(edge_attr, w, b)


def _mlp(h, agg, w1, b1, w2, b2, eps, dout, relu_out):
    return pl.pallas_call(
        functools.partial(_mlp_body, relu_out=relu_out),
        grid=(N // BN,),
        in_specs=[
            pl.BlockSpec((BN, H), lambda i: (i, 0)),
            pl.BlockSpec((BN, H), lambda i: (i, 0)),
            pl.BlockSpec((H, H), lambda i: (0, 0)),
            pl.BlockSpec((1, H), lambda i: (0, 0)),
            pl.BlockSpec((H, dout), lambda i: (0, 0)),
            pl.BlockSpec((1, dout), lambda i: (0, 0)),
            pl.BlockSpec((1, 1), lambda i: (0, 0)),
        ],
        out_specs=pl.BlockSpec((BN, dout), lambda i: (i, 0)),
        out_shape=jax.ShapeDtypeStruct((N, dout), jnp.float32),
    )(h, agg, w1, b1, w2, b2, eps)


def _sc_agg_body(h_hbm, e_hbm, src_hbm, dst_hbm, out_hbm,
                 sbuf, dbuf, dloc, grows, ebuf, aggsp, gsem, ssem):
    cid = lax.axis_index("c")
    sid = lax.axis_index("s")
    base = cid * HALF

    def zbody(i, carry):
        grows[i, pl.ds(0, 16)] = jnp.zeros((16,), jnp.float32)
        grows[i, pl.ds(16, 16)] = jnp.zeros((16,), jnp.float32)
        return carry

    lax.fori_loop(0, SUB_E, zbody, 0)
    for k in range(ZCOPIES):
        pltpu.sync_copy(grows, aggsp.at[pl.ds(sid * STRIPE + k * SUB_E, SUB_E)])
    pltpu.sync_copy(grows.at[pl.ds(0, ZREM)],
                    aggsp.at[pl.ds(sid * STRIPE + ZCOPIES * SUB_E, ZREM)])
    plsc.subcore_barrier()

    nchunks = jnp.where(sid < 10, 98, 97)
    chunk0 = 97 * sid + jnp.minimum(sid, 10)

    def compute_dloc(nrows):
        for j in range(nrows):
            for k in range(8):
                d = dbuf[j, pl.ds(k * 16, 16)]
                l = d - base
                ok = (l >= 0) & (l < HALF)
                dloc[j, pl.ds(k * 16, 16)] = jnp.where(ok, l, DUMMY)

    def do_sub(rb, j0):
        cps = [
            pltpu.async_copy(h_hbm.at[sbuf.at[j0 + j]],
                             grows.at[pl.ds(j * 128, 128)], gsem)
            for j in range(SUB_ROWS)
        ]
        pltpu.sync_copy(e_hbm.at[pl.ds((rb + j0) * 128, SUB_E)], ebuf)
        for cp in cps:
            cp.wait()

        def cbody(i, carry):
            for half in (0, 16):
                g = grows[i, pl.ds(half, 16)]
                v = jnp.maximum(g + ebuf[i, pl.ds(half, 16)], 0.0)
                grows[i, pl.ds(half, 16)] = v
            return carry

        lax.fori_loop(0, SUB_E, cbody, 0)
        scps = [
            pltpu.async_copy(grows.at[pl.ds(j * 128, 128)],
                             aggsp.at[dloc.at[j0 + j]], ssem, add=True)
            for j in range(SUB_ROWS)
        ]
        for cp in scps:
            cp.wait()

    def chunk_body(c, carry):
        rb = (chunk0 + c) * CHUNK_ROWS
        pltpu.sync_copy(src_hbm.at[pl.ds(rb, CHUNK_ROWS)], sbuf)
        pltpu.sync_copy(dst_hbm.at[pl.ds(rb, CHUNK_ROWS)], dbuf)
        compute_dloc(CHUNK_ROWS)
        for q in range(CHUNK_ROWS // SUB_ROWS):
            do_sub(rb, q * SUB_ROWS)
        return carry

    lax.fori_loop(0, nchunks, chunk_body, 0)

    @pl.when(sid == 15)
    def _tail():
        rb = NCHUNKS * CHUNK_ROWS
        pltpu.sync_copy(src_hbm.at[pl.ds(rb, TAIL_ROWS)],
                        sbuf.at[pl.ds(0, TAIL_ROWS)])
        pltpu.sync_copy(dst_hbm.at[pl.ds(rb, TAIL_ROWS)],
                        dbuf.at[pl.ds(0, TAIL_ROWS)])
        compute_dloc(TAIL_ROWS)
        for q in range(TAIL_ROWS // SUB_ROWS):
            do_sub(rb, q * SUB_ROWS)

    plsc.subcore_barrier()
    g0 = 390 * sid + jnp.minimum(sid, 10)
    pltpu.sync_copy(aggsp.at[pl.ds(g0 * 8, 3120)],
                    out_hbm.at[pl.ds(base + g0 * 8, 3120)])

    @pl.when(sid < 10)
    def _wb_extra():
        off = (g0 + 390) * 8
        pltpu.sync_copy(aggsp.at[pl.ds(off, 8)],
                        out_hbm.at[pl.ds(base + off, 8)])


def _sc_agg(h, e, src2, dst2):
    mesh = plsc.VectorSubcoreMesh(core_axis_name="c", subcore_axis_name="s",
                                  num_cores=2, num_subcores=16)
    return pl.kernel(
        _sc_agg_body,
        out_type=jax.ShapeDtypeStruct((N, H), jnp.float32),
        mesh=mesh,
        scratch_types=[
            pltpu.VMEM((CHUNK_ROWS, 128), jnp.int32),
            pltpu.VMEM((CHUNK_ROWS, 128), jnp.int32),
            pltpu.VMEM((CHUNK_ROWS, 128), jnp.int32),
            pltpu.VMEM((SUB_E, H), jnp.float32),
            pltpu.VMEM((SUB_E, H), jnp.float32),
            pltpu.VMEM_SHARED((SP_ROWS, H), jnp.float32),
            pltpu.SemaphoreType.DMA,
            pltpu.SemaphoreType.DMA,
        ],
        compiler_params=pltpu.CompilerParams(use_tc_tiling_on_sc=False),
    )(h, e, src2, dst2)


def kernel(x, edge_index, edge_attr, type_embed, feat_W, feat_b, edge_W,
           edge_b, W1_0, b1_0, W2_0, b2_0, eps_0, W1_1, b1_1, W2_1, b2_1,
           eps_1, W1_2, b1_2, W2_2, b2_2, eps_2):
    w_pad = jnp.concatenate([jnp.zeros((1, H), jnp.float32), feat_W], axis=0)
    h = _node_embed(x, type_embed, w_pad, feat_b.reshape(1, H))
    e = _edge_embed(edge_attr, edge_W, edge_b.reshape(1, H))
    src2 = edge_index[0].reshape(ROWS, 128)
    dst2 = edge_index[1].reshape(ROWS, 128)

    layers = [
        (W1_0, b1_0, W2_0, b2_0, eps_0, H, True),
        (W1_1, b1_1, W2_1, b2_1, eps_1, H, True),
        (W1_2, b1_2, W2_2, b2_2, eps_2, PE, False),
    ]
    for w1, b1, w2, b2, eps, dout, relu_out in layers:
        agg = _sc_agg(h, e, src2, dst2)
        h = _mlp(h, agg, w1, b1.reshape(1, H), w2, b2.reshape(1, dout),
                 eps.reshape(1, 1), dout, relu_out)
    return h


# speedup vs baseline: 4.0783x; 1.0506x over previous
import functools

import jax
import jax.numpy as jnp
from jax import lax
from jax.experimental import pallas as pl
from jax.experimental.pallas import tpu as pltpu
from jax.experimental.pallas import tpu_sc as plsc

N = 100000
E = 1600000
NT = 32
NF = 9
NEF = 16
H = 32
PE = 16
HH = H // 2

ROWS = E // 128
SP_ROWS = 100096
STRIPE = SP_ROWS // 16
CHUNK_ROWS = 8
SUB_ROWS = 2
SUB_E = SUB_ROWS * 128
NSUB = CHUNK_ROWS // SUB_ROWS
NCHUNKS = ROWS // CHUNK_ROWS
TAIL_ROWS = ROWS - NCHUNKS * CHUNK_ROWS
ZCOPIES = STRIPE // SUB_E
ZREM = STRIPE - ZCOPIES * SUB_E


def _node_embed_body(x_ref, te_ref, w_ref, b_ref, o_ref):
    xb = x_ref[...]
    t = xb[:, 0:1]
    iot = lax.broadcasted_iota(jnp.int32, (1, NT), 1).astype(jnp.float32)
    oh = (t == iot).astype(jnp.float32)
    h = jnp.dot(oh, te_ref[...], preferred_element_type=jnp.float32)
    h = h + jnp.dot(xb, w_ref[...], preferred_element_type=jnp.float32)
    h = h + b_ref[...]
    o_ref[0] = h[:, :HH]
    o_ref[1] = h[:, HH:]


def _edge_embed_body(ea_ref, w_ref, b_ref, o_ref):
    e = (jnp.dot(ea_ref[...], w_ref[...], preferred_element_type=jnp.float32)
         + b_ref[...])
    o_ref[0] = e[:, :HH]
    o_ref[1] = e[:, HH:]


def _mlp_body(hlo_ref, hhi_ref, alo_ref, ahi_ref, w1_ref, b1_ref, w2_ref,
              b2_ref, eps_ref, o_ref, *, relu_out, split_out):
    s = 1.0 + eps_ref[0, 0]
    z = jnp.concatenate(
        [s * hlo_ref[0] + alo_ref[0], s * hhi_ref[0] + ahi_ref[0]],
        axis=1,
    )
    z = jnp.maximum(
        jnp.dot(z, w1_ref[...], preferred_element_type=jnp.float32) + b1_ref[...],
        0.0,
    )
    out = jnp.dot(z, w2_ref[...], preferred_element_type=jnp.float32) + b2_ref[...]
    if relu_out:
        out = jnp.maximum(out, 0.0)
    if split_out:
        o_ref[0] = out[:, :HH]
        o_ref[1] = out[:, HH:]
    else:
        o_ref[...] = out


BN = 2000
BE = 8000


def _node_embed(x, type_embed, w_pad, b):
    return pl.pallas_call(
        _node_embed_body,
        grid=(N // BN,),
        in_specs=[
            pl.BlockSpec((BN, NF), lambda i: (i, 0)),
            pl.BlockSpec((NT, H), lambda i: (0, 0)),
            pl.BlockSpec((NF, H), lambda i: (0, 0)),
            pl.BlockSpec((1, H), lambda i: (0, 0)),
        ],
        out_specs=pl.BlockSpec((2, BN, HH), lambda i: (0, i, 0)),
        out_shape=jax.ShapeDtypeStruct((2, N, HH), jnp.float32),
    )(x, type_embed, w_pad, b)


def _edge_embed(edge_attr, w, b):
    return pl.pallas_call(
        _edge_embed_body,
        grid=(E // BE,),
        in_specs=[
            pl.BlockSpec((BE, NEF), lambda i: (i, 0)),
            pl.BlockSpec((NEF, H), lambda i: (0, 0)),
            pl.BlockSpec((1, H), lambda i: (0, 0)),
        ],
        out_specs=pl.BlockSpec((2, BE, HH), lambda i: (0, i, 0)),
        out_shape=jax.ShapeDtypeStruct((2, E, HH), jnp.float32),
    )(edge_attr, w, b)


def _mlp(h3, agg3, w1, b1, w2, b2, eps, dout, relu_out, split_out):
    if split_out:
        out_specs = pl.BlockSpec((2, BN, HH), lambda i: (0, i, 0))
        out_shape = jax.ShapeDtypeStruct((2, N, HH), jnp.float32)
    else:
        out_specs = pl.BlockSpec((BN, dout), lambda i: (i, 0))
        out_shape = jax.ShapeDtypeStruct((N, dout), jnp.float32)
    return pl.pallas_call(
        functools.partial(_mlp_body, relu_out=relu_out, split_out=split_out),
        grid=(N // BN,),
        in_specs=[
            pl.BlockSpec((1, BN, HH), lambda i: (0, i, 0)),
            pl.BlockSpec((1, BN, HH), lambda i: (1, i, 0)),
            pl.BlockSpec((1, BN, HH), lambda i: (0, i, 0)),
            pl.BlockSpec((1, BN, HH), lambda i: (1, i, 0)),
            pl.BlockSpec((H, H), lambda i: (0, 0)),
            pl.BlockSpec((1, H), lambda i: (0, 0)),
            pl.BlockSpec((H, dout), lambda i: (0, 0)),
            pl.BlockSpec((1, dout), lambda i: (0, 0)),
            pl.BlockSpec((1, 1), lambda i: (0, 0)),
        ],
        out_specs=out_specs,
        out_shape=out_shape,
    )(h3, h3, agg3, agg3, w1, b1, w2, b2, eps)


def _sc_agg_body(h_hbm, e_hbm, src_hbm, dst_hbm, out_hbm,
                 sbuf, soff, dbuf, g0b, g1b, e0b, e1b, aggsp,
                 gsem0, gsem1, esem0, esem1, ssem0, ssem1):
    cid = lax.axis_index("c")
    sid = lax.axis_index("s")

    gb = [g0b, g1b]
    eb = [e0b, e1b]
    gsem = [gsem0, gsem1]
    esem = [esem0, esem1]
    ssem = [ssem0, ssem1]
    hoff = cid * N
    eoff = cid * E

    def zbody(i, carry):
        g0b[i, pl.ds(0, 16)] = jnp.zeros((16,), jnp.float32)
        return carry

    lax.fori_loop(0, SUB_E, zbody, 0)
    for k in range(ZCOPIES):
        pltpu.sync_copy(g0b, aggsp.at[pl.ds(sid * STRIPE + k * SUB_E, SUB_E)])
    if ZREM:
        pltpu.sync_copy(g0b.at[pl.ds(0, ZREM)],
                        aggsp.at[pl.ds(sid * STRIPE + ZCOPIES * SUB_E, ZREM)])
    plsc.subcore_barrier()

    nchunks = jnp.where(sid < 10, 98, 97)
    chunk0 = 97 * sid + jnp.minimum(sid, 10)

    def issue_in(rb, q, slot):
        g = [
            pltpu.async_copy(
                h_hbm.at[soff.at[q * SUB_ROWS + j]],
                gb[slot].at[pl.ds(j * 128, 128)], gsem[slot])
            for j in range(SUB_ROWS)
        ]
        e = pltpu.async_copy(
            e_hbm.at[pl.ds(eoff + (rb + q * SUB_ROWS) * 128, SUB_E)],
            eb[slot], esem[slot])
        return g + [e]

    def compute(slot):
        def cbody(i, carry):
            v = gb[slot][i, pl.ds(0, 16)] + eb[slot][i, pl.ds(0, 16)]
            gb[slot][i, pl.ds(0, 16)] = jnp.maximum(v, 0.0)
            return carry

        lax.fori_loop(0, SUB_E, cbody, 0)

    def issue_scatter(q, slot):
        return [
            pltpu.async_copy(gb[slot].at[pl.ds(j * 128, 128)],
                             aggsp.at[dbuf.at[q * SUB_ROWS + j]], ssem[slot],
                             add=True)
            for j in range(SUB_ROWS)
        ]

    def load_idx(rb, nrows):
        pltpu.sync_copy(src_hbm.at[pl.ds(rb, nrows)],
                        sbuf.at[pl.ds(0, nrows)])
        pltpu.sync_copy(dst_hbm.at[pl.ds(rb, nrows)],
                        dbuf.at[pl.ds(0, nrows)])
        for j in range(nrows):
            for k in range(8):
                soff[j, pl.ds(k * 16, 16)] = sbuf[j, pl.ds(k * 16, 16)] + hoff

    def chunk_body(c, carry):
        rb = (chunk0 + c) * CHUNK_ROWS
        load_idx(rb, CHUNK_ROWS)
        cps = {0: issue_in(rb, 0, 0)}
        scs = {}
        for q in range(NSUB):
            if q - 1 in scs:
                for cp in scs.pop(q - 1):
                    cp.wait()
            if q + 1 < NSUB:
                cps[q + 1] = issue_in(rb, q + 1, (q + 1) % 2)
            for cp in cps.pop(q):
                cp.wait()
            compute(q % 2)
            scs[q] = issue_scatter(q, q % 2)
        for cp in scs.pop(NSUB - 1):
            cp.wait()
        return carry

    lax.fori_loop(0, nchunks, chunk_body, 0)

    @pl.when(sid == 15)
    def _tail():
        rb = NCHUNKS * CHUNK_ROWS
        load_idx(rb, TAIL_ROWS)
        for q in range(TAIL_ROWS // SUB_ROWS):
            for cp in issue_in(rb, q, q % 2):
                cp.wait()
            compute(q % 2)
            for cp in issue_scatter(q, q % 2):
                cp.wait()

    plsc.subcore_barrier()
    g0 = 781 * sid + jnp.minimum(sid, 4)
    pltpu.sync_copy(aggsp.at[pl.ds(g0 * 8, 6248)],
                    out_hbm.at[pl.ds(cid * N + g0 * 8, 6248)])

    @pl.when(sid < 4)
    def _wb_extra():
        off = (g0 + 781) * 8
        pltpu.sync_copy(aggsp.at[pl.ds(off, 8)],
                        out_hbm.at[pl.ds(cid * N + off, 8)])


def _sc_agg(h_cat, e_cat, src2, dst2):
    mesh = plsc.VectorSubcoreMesh(core_axis_name="c", subcore_axis_name="s",
                                  num_cores=2, num_subcores=16)
    return pl.kernel(
        _sc_agg_body,
        out_type=jax.ShapeDtypeStruct((2 * N, HH), jnp.float32),
        mesh=mesh,
        scratch_types=[
            pltpu.VMEM((CHUNK_ROWS, 128), jnp.int32),
            pltpu.VMEM((CHUNK_ROWS, 128), jnp.int32),
            pltpu.VMEM((CHUNK_ROWS, 128), jnp.int32),
            pltpu.VMEM((SUB_E, HH), jnp.float32),
            pltpu.VMEM((SUB_E, HH), jnp.float32),
            pltpu.VMEM((SUB_E, HH), jnp.float32),
            pltpu.VMEM((SUB_E, HH), jnp.float32),
            pltpu.VMEM_SHARED((SP_ROWS, HH), jnp.float32),
            pltpu.SemaphoreType.DMA,
            pltpu.SemaphoreType.DMA,
            pltpu.SemaphoreType.DMA,
            pltpu.SemaphoreType.DMA,
            pltpu.SemaphoreType.DMA,
            pltpu.SemaphoreType.DMA,
        ],
        compiler_params=pltpu.CompilerParams(use_tc_tiling_on_sc=False),
    )(h_cat, e_cat, src2, dst2)


def kernel(x, edge_index, edge_attr, type_embed, feat_W, feat_b, edge_W,
           edge_b, W1_0, b1_0, W2_0, b2_0, eps_0, W1_1, b1_1, W2_1, b2_1,
           eps_1, W1_2, b1_2, W2_2, b2_2, eps_2):
    w_pad = jnp.concatenate([jnp.zeros((1, H), jnp.float32), feat_W], axis=0)
    h3 = _node_embed(x, type_embed, w_pad, feat_b.reshape(1, H))
    e_cat = _edge_embed(edge_attr, edge_W, edge_b.reshape(1, H)).reshape(2 * E, HH)
    src2 = edge_index[0].reshape(ROWS, 128)
    dst2 = edge_index[1].reshape(ROWS, 128)

    layers = [
        (W1_0, b1_0, W2_0, b2_0, eps_0, H, True, True),
        (W1_1, b1_1, W2_1, b2_1, eps_1, H, True, True),
        (W1_2, b1_2, W2_2, b2_2, eps_2, PE, False, False),
    ]
    for w1, b1, w2, b2, eps, dout, relu_out, split_out in layers:
        agg3 = _sc_agg(h3.reshape(2 * N, HH), e_cat, src2, dst2).reshape(2, N, HH)
        h3 = _mlp(h3, agg3, w1, b1.reshape(1, H), w2, b2.reshape(1, dout),
                  eps.reshape(1, 1), dout, relu_out, split_out)
    return h3


# speedup vs baseline: 7.2192x; 1.7701x over previous
import functools

import jax
import jax.numpy as jnp
from jax import lax
from jax.experimental import pallas as pl
from jax.experimental.pallas import tpu as pltpu
from jax.experimental.pallas import tpu_sc as plsc

N = 100000
E = 1600000
NT = 32
NF = 9
NEF = 16
H = 32
PE = 16
HH = H // 2
EP = E // 8

ROWS = E // 128
SP_ROWS = 100096
STRIPE = SP_ROWS // 16
CHUNK_ROWS = 8
SUB_ROWS = 2
SUB_E = SUB_ROWS * 128
SUB_P = SUB_E // 8
NSUB = CHUNK_ROWS // SUB_ROWS
NCHUNKS = ROWS // CHUNK_ROWS
TAIL_ROWS = ROWS - NCHUNKS * CHUNK_ROWS
ZCOPIES = STRIPE // SUB_E
ZREM = STRIPE - ZCOPIES * SUB_E

BN = 2000
NBLK = N // BN
BEP = 8000
EBLK = EP // BEP


def _node_embed_body(x_ref, te_ref, w_ref, b_ref, o_ref):
    xb = x_ref[...]
    t = xb[:, 0:1]
    iot = lax.broadcasted_iota(jnp.int32, (1, NT), 1).astype(jnp.float32)
    oh = (t == iot).astype(jnp.float32)
    h = jnp.dot(oh, te_ref[0], preferred_element_type=jnp.float32)
    h = h + jnp.dot(xb, w_ref[0], preferred_element_type=jnp.float32)
    o_ref[...] = h + b_ref[0]


def _edge_embed_body(ea_ref, wk_ref, b_ref, o_ref):
    o_ref[...] = (
        jnp.dot(ea_ref[...], wk_ref[0], preferred_element_type=jnp.float32)
        + b_ref[0]
    )


def _mlp_body(hlo_ref, hhi_ref, alo_ref, ahi_ref, w1_ref, b1_ref, w2_ref,
              b2_ref, eps_ref, o_ref, *, relu_out, split_out):
    s = 1.0 + eps_ref[0, 0]
    z = jnp.concatenate(
        [s * hlo_ref[...] + alo_ref[...], s * hhi_ref[...] + ahi_ref[...]],
        axis=1,
    )
    z = jnp.maximum(
        jnp.dot(z, w1_ref[...], preferred_element_type=jnp.float32) + b1_ref[...],
        0.0,
    )
    w2 = w2_ref[0] if split_out else w2_ref[...]
    b2 = b2_ref[0] if split_out else b2_ref[...]
    out = jnp.dot(z, w2, preferred_element_type=jnp.float32) + b2
    if relu_out:
        out = jnp.maximum(out, 0.0)
    o_ref[...] = out


def _node_embed(x, type_embed, w_pad, b):
    return pl.pallas_call(
        _node_embed_body,
        grid=(2, NBLK),
        in_specs=[
            pl.BlockSpec((BN, NF), lambda c, i: (i, 0)),
            pl.BlockSpec((1, NT, HH), lambda c, i: (c, 0, 0)),
            pl.BlockSpec((1, NF, HH), lambda c, i: (c, 0, 0)),
            pl.BlockSpec((1, 1, HH), lambda c, i: (c, 0, 0)),
        ],
        out_specs=pl.BlockSpec((BN, HH), lambda c, i: (c * NBLK + i, 0)),
        out_shape=jax.ShapeDtypeStruct((2 * N, HH), jnp.float32),
    )(x, type_embed, w_pad, b)


def _edge_embed(ea_packed, wk, b_tile):
    return pl.pallas_call(
        _edge_embed_body,
        grid=(2, EBLK),
        in_specs=[
            pl.BlockSpec((BEP, 128), lambda c, i: (i, 0)),
            pl.BlockSpec((1, 128, 128), lambda c, i: (c, 0, 0)),
            pl.BlockSpec((1, 1, 128), lambda c, i: (c, 0, 0)),
        ],
        out_specs=pl.BlockSpec((BEP, 128), lambda c, i: (c * EBLK + i, 0)),
        out_shape=jax.ShapeDtypeStruct((2 * EP, 128), jnp.float32),
    )(ea_packed, wk, b_tile)


def _mlp(h, agg, w1, b1, w2, b2, eps, relu_out, split_out):
    if split_out:
        grid = (2, NBLK)
        w2_spec = pl.BlockSpec((1, H, HH), lambda c, i: (c, 0, 0))
        b2_spec = pl.BlockSpec((1, 1, HH), lambda c, i: (c, 0, 0))
        out_specs = pl.BlockSpec((BN, HH), lambda c, i: (c * NBLK + i, 0))
        out_shape = jax.ShapeDtypeStruct((2 * N, HH), jnp.float32)
        lo = lambda c, i: (i, 0)
        hi = lambda c, i: (NBLK + i, 0)
        fixed = lambda c, i: (0, 0)
    else:
        grid = (NBLK,)
        w2_spec = pl.BlockSpec((H, PE), lambda i: (0, 0))
        b2_spec = pl.BlockSpec((1, PE), lambda i: (0, 0))
        out_specs = pl.BlockSpec((BN, PE), lambda i: (i, 0))
        out_shape = jax.ShapeDtypeStruct((N, PE), jnp.float32)
        lo = lambda i: (i, 0)
        hi = lambda i: (NBLK + i, 0)
        fixed = lambda i: (0, 0)
    return pl.pallas_call(
        functools.partial(_mlp_body, relu_out=relu_out, split_out=split_out),
        grid=grid,
        in_specs=[
            pl.BlockSpec((BN, HH), lo),
            pl.BlockSpec((BN, HH), hi),
            pl.BlockSpec((BN, HH), lo),
            pl.BlockSpec((BN, HH), hi),
            pl.BlockSpec((H, H), fixed),
            pl.BlockSpec((1, H), fixed),
            w2_spec,
            b2_spec,
            pl.BlockSpec((1, 1), fixed),
        ],
        out_specs=out_specs,
        out_shape=out_shape,
    )(h, h, agg, agg, w1, b1, w2, b2, eps)


def _sc_agg_body(h_hbm, e_hbm, src_hbm, dst_hbm, out_hbm,
                 sbuf, soff, dbuf, g0b, g1b, e0b, e1b, aggsp,
                 gsem0, gsem1, esem0, esem1, ssem0, ssem1):
    cid = lax.axis_index("c")
    sid = lax.axis_index("s")

    gb = [g0b, g1b]
    eb = [e0b, e1b]
    gsem = [gsem0, gsem1]
    esem = [esem0, esem1]
    ssem = [ssem0, ssem1]
    hoff = cid * N
    eoff = cid * EP

    @plsc.parallel_loop(0, SUB_E, unroll=8)
    def _zero(i):
        g0b[i, pl.ds(0, 16)] = jnp.zeros((16,), jnp.float32)

    for k in range(ZCOPIES):
        pltpu.sync_copy(g0b, aggsp.at[pl.ds(sid * STRIPE + k * SUB_E, SUB_E)])
    if ZREM:
        pltpu.sync_copy(g0b.at[pl.ds(0, ZREM)],
                        aggsp.at[pl.ds(sid * STRIPE + ZCOPIES * SUB_E, ZREM)])
    plsc.subcore_barrier()

    nchunks = jnp.where(sid < 10, 98, 97)
    chunk0 = 97 * sid + jnp.minimum(sid, 10)

    def issue_in(rb, q, slot):
        g = [
            pltpu.async_copy(
                h_hbm.at[soff.at[q * SUB_ROWS + j]],
                gb[slot].at[pl.ds(j * 128, 128)], gsem[slot])
            for j in range(SUB_ROWS)
        ]
        e = pltpu.async_copy(
            e_hbm.at[pl.ds(eoff + (rb + q * SUB_ROWS) * 16, SUB_P)],
            eb[slot], esem[slot])
        return g + [e]

    def compute(slot):
        gref, eref = gb[slot], eb[slot]

        @plsc.parallel_loop(0, SUB_P, unroll=2)
        def _relu_add(i):
            for k in range(8):
                v = gref[i * 8 + k, pl.ds(0, 16)] + eref[i, pl.ds(k * 16, 16)]
                gref[i * 8 + k, pl.ds(0, 16)] = jnp.maximum(v, 0.0)

    def issue_scatter(q, slot):
        return [
            pltpu.async_copy(gb[slot].at[pl.ds(j * 128, 128)],
                             aggsp.at[dbuf.at[q * SUB_ROWS + j]], ssem[slot],
                             add=True)
            for j in range(SUB_ROWS)
        ]

    def load_idx(rb, nrows):
        pltpu.sync_copy(src_hbm.at[pl.ds(rb, nrows)],
                        sbuf.at[pl.ds(0, nrows)])
        pltpu.sync_copy(dst_hbm.at[pl.ds(rb, nrows)],
                        dbuf.at[pl.ds(0, nrows)])
        for j in range(nrows):
            for k in range(8):
                soff[j, pl.ds(k * 16, 16)] = sbuf[j, pl.ds(k * 16, 16)] + hoff

    def chunk_body(c, carry):
        rb = (chunk0 + c) * CHUNK_ROWS
        load_idx(rb, CHUNK_ROWS)
        cps = {0: issue_in(rb, 0, 0)}
        scs = {}
        for q in range(NSUB):
            if q - 1 in scs:
                for cp in scs.pop(q - 1):
                    cp.wait()
            if q + 1 < NSUB:
                cps[q + 1] = issue_in(rb, q + 1, (q + 1) % 2)
            for cp in cps.pop(q):
                cp.wait()
            compute(q % 2)
            scs[q] = issue_scatter(q, q % 2)
        for cp in scs.pop(NSUB - 1):
            cp.wait()
        return carry

    lax.fori_loop(0, nchunks, chunk_body, 0)

    @pl.when(sid == 15)
    def _tail():
        rb = NCHUNKS * CHUNK_ROWS
        load_idx(rb, TAIL_ROWS)
        for q in range(TAIL_ROWS // SUB_ROWS):
            for cp in issue_in(rb, q, q % 2):
                cp.wait()
            compute(q % 2)
            for cp in issue_scatter(q, q % 2):
                cp.wait()

    plsc.subcore_barrier()
    g0 = 781 * sid + jnp.minimum(sid, 4)
    pltpu.sync_copy(aggsp.at[pl.ds(g0 * 8, 6248)],
                    out_hbm.at[pl.ds(cid * N + g0 * 8, 6248)])

    @pl.when(sid < 4)
    def _wb_extra():
        off = (g0 + 781) * 8
        pltpu.sync_copy(aggsp.at[pl.ds(off, 8)],
                        out_hbm.at[pl.ds(cid * N + off, 8)])


def _sc_agg(h_cat, e_packed, src2, dst2):
    mesh = plsc.VectorSubcoreMesh(core_axis_name="c", subcore_axis_name="s",
                                  num_cores=2, num_subcores=16)
    return pl.kernel(
        _sc_agg_body,
        out_type=jax.ShapeDtypeStruct((2 * N, HH), jnp.float32),
        mesh=mesh,
        scratch_types=[
            pltpu.VMEM((CHUNK_ROWS, 128), jnp.int32),
            pltpu.VMEM((CHUNK_ROWS, 128), jnp.int32),
            pltpu.VMEM((CHUNK_ROWS, 128), jnp.int32),
            pltpu.VMEM((SUB_E, HH), jnp.float32),
            pltpu.VMEM((SUB_E, HH), jnp.float32),
            pltpu.VMEM((SUB_P, 128), jnp.float32),
            pltpu.VMEM((SUB_P, 128), jnp.float32),
            pltpu.VMEM_SHARED((SP_ROWS, HH), jnp.float32),
            pltpu.SemaphoreType.DMA,
            pltpu.SemaphoreType.DMA,
            pltpu.SemaphoreType.DMA,
            pltpu.SemaphoreType.DMA,
            pltpu.SemaphoreType.DMA,
            pltpu.SemaphoreType.DMA,
        ],
        compiler_params=pltpu.CompilerParams(use_tc_tiling_on_sc=False),
    )(h_cat, e_packed, src2, dst2)


def kernel(x, edge_index, edge_attr, type_embed, feat_W, feat_b, edge_W,
           edge_b, W1_0, b1_0, W2_0, b2_0, eps_0, W1_1, b1_1, W2_1, b2_1,
           eps_1, W1_2, b1_2, W2_2, b2_2, eps_2):
    w_pad = jnp.concatenate([jnp.zeros((1, H), jnp.float32), feat_W], axis=0)

    def col_halves(m):
        return jnp.stack([m[:, :HH], m[:, HH:]])

    h = _node_embed(x, col_halves(type_embed), col_halves(w_pad),
                    col_halves(feat_b.reshape(1, H)).reshape(2, 1, HH))

    ea_packed = edge_attr.reshape(EP, 128)
    eye8 = jnp.eye(8, dtype=jnp.float32)
    wk = jnp.stack([jnp.kron(eye8, edge_W[:, :HH]),
                    jnp.kron(eye8, edge_W[:, HH:])])
    b_tile = jnp.stack([jnp.tile(edge_b[:HH], 8),
                        jnp.tile(edge_b[HH:], 8)]).reshape(2, 1, 128)
    e_packed = _edge_embed(ea_packed, wk, b_tile)

    src2 = edge_index[0].reshape(ROWS, 128)
    dst2 = edge_index[1].reshape(ROWS, 128)

    layers = [
        (W1_0, b1_0, W2_0, b2_0, eps_0, True, True),
        (W1_1, b1_1, W2_1, b2_1, eps_1, True, True),
        (W1_2, b1_2, W2_2, b2_2, eps_2, False, False),
    ]
    for w1, b1, w2, b2, eps, relu_out, split_out in layers:
        agg = _sc_agg(h, e_packed, src2, dst2)
        if split_out:
            w2a = col_halves(w2)
            b2a = col_halves(b2.reshape(1, H)).reshape(2, 1, HH)
        else:
            w2a = w2
            b2a = b2.reshape(1, PE)
        h = _mlp(h, agg, w1, b1.reshape(1, H), w2a, b2a, eps.reshape(1, 1),
                 relu_out, split_out)
    return h


# speedup vs baseline: 9.8274x; 1.3613x over previous
import functools

import jax
import jax.numpy as jnp
from jax import lax
from jax.experimental import pallas as pl
from jax.experimental.pallas import tpu as pltpu
from jax.experimental.pallas import tpu_sc as plsc

N = 100000
E = 1600000
NT = 32
NF = 9
NEF = 16
H = 32
PE = 16
HH = H // 2
EP = E // 8

ROWS = E // 128
SP_ROWS = 100096
STRIPE = SP_ROWS // 16
CHUNK_ROWS = 8
SUB_ROWS = 2
SUB_E = SUB_ROWS * 128
SUB_P = SUB_E // 8
NSUB = CHUNK_ROWS // SUB_ROWS
NCHUNKS = ROWS // CHUNK_ROWS
TAIL_ROWS = ROWS - NCHUNKS * CHUNK_ROWS
ZCOPIES = STRIPE // SUB_E
ZREM = STRIPE - ZCOPIES * SUB_E

BN = 2000
NBLK = N // BN
NPB = N // 8
BNP = BN // 8
BEP = 8000
EBLK = EP // BEP


def _node_embed_body(x_ref, te_ref, w_ref, b_ref, o_ref):
    xb = x_ref[...]
    t = xb[:, 0:1]
    iot = lax.broadcasted_iota(jnp.int32, (1, NT), 1).astype(jnp.float32)
    oh = (t == iot).astype(jnp.float32)
    h = jnp.dot(oh, te_ref[0], preferred_element_type=jnp.float32)
    h = h + jnp.dot(xb, w_ref[0], preferred_element_type=jnp.float32)
    o_ref[...] = h + b_ref[0]


def _edge_embed_body(ea_ref, wk_ref, b_ref, o_ref):
    o_ref[...] = (
        jnp.dot(ea_ref[...], wk_ref[0], preferred_element_type=jnp.float32)
        + b_ref[0]
    )


def _mlp_body(hlo_ref, hhi_ref, alo_ref, ahi_ref, k1lo_ref, k1hi_ref,
              b1_ref, k2_ref, b2_ref, eps_ref, o_ref, *, relu_out, split_out):
    s = 1.0 + eps_ref[0, 0]
    zlo = s * hlo_ref[0] + alo_ref[0]
    zhi = s * hhi_ref[0] + ahi_ref[0]
    z = (jnp.dot(zlo, k1lo_ref[...], preferred_element_type=jnp.float32)
         + jnp.dot(zhi, k1hi_ref[...], preferred_element_type=jnp.float32)
         + b1_ref[...])
    z = jnp.maximum(z, 0.0)
    k2 = k2_ref[0] if split_out else k2_ref[...]
    b2 = b2_ref[0] if split_out else b2_ref[...]
    out = jnp.dot(z, k2, preferred_element_type=jnp.float32) + b2
    if relu_out:
        out = jnp.maximum(out, 0.0)
    if split_out:
        o_ref[0] = out
    else:
        o_ref[...] = out


def _node_embed(x, type_embed, w_pad, b):
    return pl.pallas_call(
        _node_embed_body,
        grid=(2, NBLK),
        in_specs=[
            pl.BlockSpec((BN, NF), lambda c, i: (i, 0)),
            pl.BlockSpec((1, NT, HH), lambda c, i: (c, 0, 0)),
            pl.BlockSpec((1, NF, HH), lambda c, i: (c, 0, 0)),
            pl.BlockSpec((1, 1, HH), lambda c, i: (c, 0, 0)),
        ],
        out_specs=pl.BlockSpec((BN, HH), lambda c, i: (c * NBLK + i, 0)),
        out_shape=jax.ShapeDtypeStruct((2 * N, HH), jnp.float32),
    )(x, type_embed, w_pad, b)


def _edge_embed(ea_packed, wk, b_tile):
    return pl.pallas_call(
        _edge_embed_body,
        grid=(2, EBLK),
        in_specs=[
            pl.BlockSpec((BEP, 128), lambda c, i: (i, 0)),
            pl.BlockSpec((1, 128, 128), lambda c, i: (c, 0, 0)),
            pl.BlockSpec((1, 1, 128), lambda c, i: (c, 0, 0)),
        ],
        out_specs=pl.BlockSpec((BEP, 128), lambda c, i: (c * EBLK + i, 0)),
        out_shape=jax.ShapeDtypeStruct((2 * EP, 128), jnp.float32),
    )(ea_packed, wk, b_tile)


def _mlp(hp, aggp, k1lo, k1hi, b1t, k2, b2t, eps, relu_out, split_out):
    if split_out:
        grid = (2,)
        k2_spec = pl.BlockSpec((1, 256, 128), lambda c: (c, 0, 0))
        b2_spec = pl.BlockSpec((1, 1, 128), lambda c: (c, 0, 0))
        out_specs = pl.BlockSpec((1, NPB, 128), lambda c: (c, 0, 0))
        out_shape = jax.ShapeDtypeStruct((2, NPB, 128), jnp.float32)
    else:
        grid = (1,)
        k2_spec = pl.BlockSpec((256, 128), lambda c: (0, 0))
        b2_spec = pl.BlockSpec((1, 128), lambda c: (0, 0))
        out_specs = pl.BlockSpec((NPB, 128), lambda c: (0, 0))
        out_shape = jax.ShapeDtypeStruct((NPB, 128), jnp.float32)
    lo = lambda c: (0, 0, 0)
    hi = lambda c: (1, 0, 0)
    fixed = lambda c: (0, 0)
    return pl.pallas_call(
        functools.partial(_mlp_body, relu_out=relu_out, split_out=split_out),
        grid=grid,
        in_specs=[
            pl.BlockSpec((1, NPB, 128), lo),
            pl.BlockSpec((1, NPB, 128), hi),
            pl.BlockSpec((1, NPB, 128), lo),
            pl.BlockSpec((1, NPB, 128), hi),
            pl.BlockSpec((128, 256), fixed),
            pl.BlockSpec((128, 256), fixed),
            pl.BlockSpec((1, 256), fixed),
            k2_spec,
            b2_spec,
            pl.BlockSpec((1, 1), fixed),
        ],
        out_specs=out_specs,
        out_shape=out_shape,
    )(hp, hp, aggp, aggp, k1lo, k1hi, b1t, k2, b2t, eps)


def _sc_agg_body(h_hbm, e_hbm, src_hbm, dst_hbm, out_hbm,
                 sbuf, dbuf, g0b, g1b, e0b, e1b, aggsp,
                 gsem0, gsem1, esem0, esem1, ssem0, ssem1):
    cid = lax.axis_index("c")
    sid = lax.axis_index("s")

    gb = [g0b, g1b]
    eb = [e0b, e1b]
    gsem = [gsem0, gsem1]
    esem = [esem0, esem1]
    ssem = [ssem0, ssem1]
    eoff = cid * EP

    @plsc.parallel_loop(0, SUB_E, unroll=8)
    def _zero(i):
        g0b[i, pl.ds(0, 16)] = jnp.zeros((16,), jnp.float32)

    for k in range(ZCOPIES):
        pltpu.sync_copy(g0b, aggsp.at[pl.ds(sid * STRIPE + k * SUB_E, SUB_E)])
    if ZREM:
        pltpu.sync_copy(g0b.at[pl.ds(0, ZREM)],
                        aggsp.at[pl.ds(sid * STRIPE + ZCOPIES * SUB_E, ZREM)])
    plsc.subcore_barrier()

    nchunks = jnp.where(sid < 10, 98, 97)
    chunk0 = 97 * sid + jnp.minimum(sid, 10)

    def issue_in(rb, q, slot):
        g = [
            pltpu.async_copy(
                h_hbm.at[cid].at[sbuf.at[q * SUB_ROWS + j]],
                gb[slot].at[pl.ds(j * 128, 128)], gsem[slot])
            for j in range(SUB_ROWS)
        ]
        e = pltpu.async_copy(
            e_hbm.at[pl.ds(eoff + (rb + q * SUB_ROWS) * 16, SUB_P)],
            eb[slot], esem[slot])
        return g + [e]

    def compute(slot):
        gref, eref = gb[slot], eb[slot]

        @plsc.parallel_loop(0, SUB_P, unroll=2)
        def _relu_add(i):
            for k in range(8):
                v = gref[i * 8 + k, pl.ds(0, 16)] + eref[i, pl.ds(k * 16, 16)]
                gref[i * 8 + k, pl.ds(0, 16)] = jnp.maximum(v, 0.0)

    def issue_scatter(q, slot):
        return [
            pltpu.async_copy(gb[slot].at[pl.ds(j * 128, 128)],
                             aggsp.at[dbuf.at[q * SUB_ROWS + j]], ssem[slot],
                             add=True)
            for j in range(SUB_ROWS)
        ]

    def load_idx(rb, nrows):
        pltpu.sync_copy(src_hbm.at[pl.ds(rb, nrows)],
                        sbuf.at[pl.ds(0, nrows)])
        pltpu.sync_copy(dst_hbm.at[pl.ds(rb, nrows)],
                        dbuf.at[pl.ds(0, nrows)])

    def chunk_body(c, carry):
        rb = (chunk0 + c) * CHUNK_ROWS
        load_idx(rb, CHUNK_ROWS)
        cps = {0: issue_in(rb, 0, 0)}
        scs = {}
        for q in range(NSUB):
            if q - 1 in scs:
                for cp in scs.pop(q - 1):
                    cp.wait()
            if q + 1 < NSUB:
                cps[q + 1] = issue_in(rb, q + 1, (q + 1) % 2)
            for cp in cps.pop(q):
                cp.wait()
            compute(q % 2)
            scs[q] = issue_scatter(q, q % 2)
        for cp in scs.pop(NSUB - 1):
            cp.wait()
        return carry

    lax.fori_loop(0, nchunks, chunk_body, 0)

    @pl.when(sid == 15)
    def _tail():
        rb = NCHUNKS * CHUNK_ROWS
        load_idx(rb, TAIL_ROWS)
        for q in range(TAIL_ROWS // SUB_ROWS):
            for cp in issue_in(rb, q, q % 2):
                cp.wait()
            compute(q % 2)
            for cp in issue_scatter(q, q % 2):
                cp.wait()

    plsc.subcore_barrier()
    g0 = 781 * sid + jnp.minimum(sid, 4)
    pltpu.sync_copy(aggsp.at[pl.ds(g0 * 8, 6248)],
                    out_hbm.at[pl.ds(cid * N + g0 * 8, 6248)])

    @pl.when(sid < 4)
    def _wb_extra():
        off = (g0 + 781) * 8
        pltpu.sync_copy(aggsp.at[pl.ds(off, 8)],
                        out_hbm.at[pl.ds(cid * N + off, 8)])


def _sc_agg(h_cat, e_packed, src2, dst2):
    h_cat = h_cat.reshape(2, N, HH)
    mesh = plsc.VectorSubcoreMesh(core_axis_name="c", subcore_axis_name="s",
                                  num_cores=2, num_subcores=16)
    return pl.kernel(
        _sc_agg_body,
        out_type=jax.ShapeDtypeStruct((2 * N, HH), jnp.float32),
        mesh=mesh,
        scratch_types=[
            pltpu.VMEM((CHUNK_ROWS, 128), jnp.int32),
            pltpu.VMEM((CHUNK_ROWS, 128), jnp.int32),
            pltpu.VMEM((SUB_E, HH), jnp.float32),
            pltpu.VMEM((SUB_E, HH), jnp.float32),
            pltpu.VMEM((SUB_P, 128), jnp.float32),
            pltpu.VMEM((SUB_P, 128), jnp.float32),
            pltpu.VMEM_SHARED((SP_ROWS, HH), jnp.float32),
            pltpu.SemaphoreType.DMA,
            pltpu.SemaphoreType.DMA,
            pltpu.SemaphoreType.DMA,
            pltpu.SemaphoreType.DMA,
            pltpu.SemaphoreType.DMA,
            pltpu.SemaphoreType.DMA,
        ],
        compiler_params=pltpu.CompilerParams(use_tc_tiling_on_sc=False),
    )(h_cat, e_packed, src2, dst2)


def kernel(x, edge_index, edge_attr, type_embed, feat_W, feat_b, edge_W,
           edge_b, W1_0, b1_0, W2_0, b2_0, eps_0, W1_1, b1_1, W2_1, b2_1,
           eps_1, W1_2, b1_2, W2_2, b2_2, eps_2):
    w_pad = jnp.concatenate([jnp.zeros((1, H), jnp.float32), feat_W], axis=0)

    def col_halves(m):
        return jnp.stack([m[:, :HH], m[:, HH:]])

    h = _node_embed(x, col_halves(type_embed), col_halves(w_pad),
                    col_halves(feat_b.reshape(1, H)).reshape(2, 1, HH))

    ea_packed = edge_attr.reshape(EP, 128)
    eye8 = jnp.eye(8, dtype=jnp.float32)
    wk = jnp.stack([jnp.kron(eye8, edge_W[:, :HH]),
                    jnp.kron(eye8, edge_W[:, HH:])])
    b_tile = jnp.stack([jnp.tile(edge_b[:HH], 8),
                        jnp.tile(edge_b[HH:], 8)]).reshape(2, 1, 128)
    e_packed = _edge_embed(ea_packed, wk, b_tile)

    src2 = edge_index[0].reshape(ROWS, 128)
    dst2 = edge_index[1].reshape(ROWS, 128)

    layers = [
        (W1_0, b1_0, W2_0, b2_0, eps_0, True, True),
        (W1_1, b1_1, W2_1, b2_1, eps_1, True, True),
        (W1_2, b1_2, W2_2, b2_2, eps_2, False, False),
    ]
    for w1, b1, w2, b2, eps, relu_out, split_out in layers:
        agg = _sc_agg(h, e_packed, src2, dst2)
        k1lo = jnp.kron(eye8, w1[:HH, :])
        k1hi = jnp.kron(eye8, w1[HH:, :])
        b1t = jnp.tile(b1, 8).reshape(1, 256)
        if split_out:
            k2 = jnp.stack([jnp.kron(eye8, w2[:, :HH]),
                            jnp.kron(eye8, w2[:, HH:])])
            b2t = jnp.stack([jnp.tile(b2[:HH], 8),
                             jnp.tile(b2[HH:], 8)]).reshape(2, 1, 128)
        else:
            k2 = jnp.kron(eye8, w2)
            b2t = jnp.tile(b2, 8).reshape(1, 128)
        out = _mlp(h.reshape(2, NPB, 128), agg.reshape(2, NPB, 128),
                   k1lo, k1hi, b1t, k2, b2t, eps.reshape(1, 1),
                   relu_out, split_out)
        h = out.reshape(2 * N, HH) if split_out else out.reshape(N, PE)
    return h


# speedup vs baseline: 9.8316x; 1.0004x over previous
import functools

import jax
import jax.numpy as jnp
from jax import lax
from jax.experimental import pallas as pl
from jax.experimental.pallas import tpu as pltpu
from jax.experimental.pallas import tpu_sc as plsc

N = 100000
E = 1600000
NT = 32
NF = 9
NEF = 16
H = 32
PE = 16
HH = H // 2
EP = E // 8

ROWS = E // 128
SP_ROWS = 100096
STRIPE = SP_ROWS // 16
CHUNK_ROWS = 16
SUB_ROWS = 2
SUB_E = SUB_ROWS * 128
SUB_P = SUB_E // 8
NSUB = CHUNK_ROWS // SUB_ROWS
NCHUNKS = ROWS // CHUNK_ROWS
TAIL_ROWS = ROWS - NCHUNKS * CHUNK_ROWS
ZCOPIES = STRIPE // SUB_E
ZREM = STRIPE - ZCOPIES * SUB_E

BN = 2000
NBLK = N // BN
NPB = N // 8
BNP = BN // 8
BEP = 8000
EBLK = EP // BEP


def _node_embed_body(x_ref, te_ref, w_ref, b_ref, o_ref):
    xb = x_ref[...]
    t = xb[:, 0:1]
    iot = lax.broadcasted_iota(jnp.int32, (1, NT), 1).astype(jnp.float32)
    oh = (t == iot).astype(jnp.float32)
    h = jnp.dot(oh, te_ref[0], preferred_element_type=jnp.float32)
    h = h + jnp.dot(xb, w_ref[0], preferred_element_type=jnp.float32)
    o_ref[...] = h + b_ref[0]


def _edge_embed_body(ea_ref, wk_ref, b_ref, o_ref):
    o_ref[...] = (
        jnp.dot(ea_ref[...], wk_ref[0], preferred_element_type=jnp.float32)
        + b_ref[0]
    )


def _mlp_body(hlo_ref, hhi_ref, alo_ref, ahi_ref, k1lo_ref, k1hi_ref,
              b1_ref, k2_ref, b2_ref, eps_ref, o_ref, *, relu_out, split_out):
    s = 1.0 + eps_ref[0, 0]
    zlo = s * hlo_ref[0] + alo_ref[0]
    zhi = s * hhi_ref[0] + ahi_ref[0]
    z = (jnp.dot(zlo, k1lo_ref[...], preferred_element_type=jnp.float32)
         + jnp.dot(zhi, k1hi_ref[...], preferred_element_type=jnp.float32)
         + b1_ref[...])
    z = jnp.maximum(z, 0.0)
    k2 = k2_ref[0] if split_out else k2_ref[...]
    b2 = b2_ref[0] if split_out else b2_ref[...]
    out = jnp.dot(z, k2, preferred_element_type=jnp.float32) + b2
    if relu_out:
        out = jnp.maximum(out, 0.0)
    if split_out:
        o_ref[0] = out
    else:
        o_ref[...] = out


def _node_embed(x, type_embed, w_pad, b):
    return pl.pallas_call(
        _node_embed_body,
        grid=(2, NBLK),
        in_specs=[
            pl.BlockSpec((BN, NF), lambda c, i: (i, 0)),
            pl.BlockSpec((1, NT, HH), lambda c, i: (c, 0, 0)),
            pl.BlockSpec((1, NF, HH), lambda c, i: (c, 0, 0)),
            pl.BlockSpec((1, 1, HH), lambda c, i: (c, 0, 0)),
        ],
        out_specs=pl.BlockSpec((BN, HH), lambda c, i: (c * NBLK + i, 0)),
        out_shape=jax.ShapeDtypeStruct((2 * N, HH), jnp.float32),
    )(x, type_embed, w_pad, b)


def _edge_embed(ea_packed, wk, b_tile):
    return pl.pallas_call(
        _edge_embed_body,
        grid=(2, EBLK),
        in_specs=[
            pl.BlockSpec((BEP, 128), lambda c, i: (i, 0)),
            pl.BlockSpec((1, 128, 128), lambda c, i: (c, 0, 0)),
            pl.BlockSpec((1, 1, 128), lambda c, i: (c, 0, 0)),
        ],
        out_specs=pl.BlockSpec((BEP, 128), lambda c, i: (c * EBLK + i, 0)),
        out_shape=jax.ShapeDtypeStruct((2 * EP, 128), jnp.float32),
    )(ea_packed, wk, b_tile)


def _mlp(hp, aggp, k1lo, k1hi, b1t, k2, b2t, eps, relu_out, split_out):
    if split_out:
        grid = (2,)
        k2_spec = pl.BlockSpec((1, 256, 128), lambda c: (c, 0, 0))
        b2_spec = pl.BlockSpec((1, 1, 128), lambda c: (c, 0, 0))
        out_specs = pl.BlockSpec((1, NPB, 128), lambda c: (c, 0, 0))
        out_shape = jax.ShapeDtypeStruct((2, NPB, 128), jnp.float32)
    else:
        grid = (1,)
        k2_spec = pl.BlockSpec((256, 128), lambda c: (0, 0))
        b2_spec = pl.BlockSpec((1, 128), lambda c: (0, 0))
        out_specs = pl.BlockSpec((NPB, 128), lambda c: (0, 0))
        out_shape = jax.ShapeDtypeStruct((NPB, 128), jnp.float32)
    lo = lambda c: (0, 0, 0)
    hi = lambda c: (1, 0, 0)
    fixed = lambda c: (0, 0)
    return pl.pallas_call(
        functools.partial(_mlp_body, relu_out=relu_out, split_out=split_out),
        grid=grid,
        in_specs=[
            pl.BlockSpec((1, NPB, 128), lo),
            pl.BlockSpec((1, NPB, 128), hi),
            pl.BlockSpec((1, NPB, 128), lo),
            pl.BlockSpec((1, NPB, 128), hi),
            pl.BlockSpec((128, 256), fixed),
            pl.BlockSpec((128, 256), fixed),
            pl.BlockSpec((1, 256), fixed),
            k2_spec,
            b2_spec,
            pl.BlockSpec((1, 1), fixed),
        ],
        out_specs=out_specs,
        out_shape=out_shape,
    )(hp, hp, aggp, aggp, k1lo, k1hi, b1t, k2, b2t, eps)


def _sc_agg_body(h_hbm, e_hbm, src_hbm, dst_hbm, out_hbm,
                 sbuf, dbuf, g0b, g1b, e0b, e1b, aggsp,
                 gsem0, gsem1, esem0, esem1, ssem0, ssem1):
    cid = lax.axis_index("c")
    sid = lax.axis_index("s")

    gb = [g0b, g1b]
    eb = [e0b, e1b]
    gsem = [gsem0, gsem1]
    esem = [esem0, esem1]
    ssem = [ssem0, ssem1]
    eoff = cid * EP

    @plsc.parallel_loop(0, SUB_E, unroll=8)
    def _zero(i):
        g0b[i, pl.ds(0, 16)] = jnp.zeros((16,), jnp.float32)

    zcps = [
        pltpu.async_copy(g0b, aggsp.at[pl.ds(sid * STRIPE + k * SUB_E, SUB_E)],
                         gsem0)
        for k in range(ZCOPIES)
    ]
    if ZREM:
        zcps.append(
            pltpu.async_copy(g0b.at[pl.ds(0, ZREM)],
                             aggsp.at[pl.ds(sid * STRIPE + ZCOPIES * SUB_E,
                                            ZREM)], gsem0))
    for cp in zcps:
        cp.wait()
    plsc.subcore_barrier()

    nchunks = jnp.where(sid < 13, 49, 48)
    chunk0 = 48 * sid + jnp.minimum(sid, 13)

    def issue_in(rb, q, slot):
        g = [
            pltpu.async_copy(
                h_hbm.at[cid].at[sbuf.at[q * SUB_ROWS + j]],
                gb[slot].at[pl.ds(j * 128, 128)], gsem[slot])
            for j in range(SUB_ROWS)
        ]
        e = pltpu.async_copy(
            e_hbm.at[pl.ds(eoff + (rb + q * SUB_ROWS) * 16, SUB_P)],
            eb[slot], esem[slot])
        return g + [e]

    def compute(slot):
        gref, eref = gb[slot], eb[slot]

        @plsc.parallel_loop(0, SUB_P, unroll=2)
        def _relu_add(i):
            for k in range(8):
                v = gref[i * 8 + k, pl.ds(0, 16)] + eref[i, pl.ds(k * 16, 16)]
                gref[i * 8 + k, pl.ds(0, 16)] = jnp.maximum(v, 0.0)

    def issue_scatter(q, slot):
        return [
            pltpu.async_copy(gb[slot].at[pl.ds(j * 128, 128)],
                             aggsp.at[dbuf.at[q * SUB_ROWS + j]], ssem[slot],
                             add=True)
            for j in range(SUB_ROWS)
        ]

    def load_idx(rb, nrows):
        pltpu.sync_copy(src_hbm.at[pl.ds(rb, nrows)],
                        sbuf.at[pl.ds(0, nrows)])
        pltpu.sync_copy(dst_hbm.at[pl.ds(rb, nrows)],
                        dbuf.at[pl.ds(0, nrows)])

    def chunk_body(c, carry):
        rb = (chunk0 + c) * CHUNK_ROWS
        load_idx(rb, CHUNK_ROWS)
        cps = {0: issue_in(rb, 0, 0)}
        scs = {}
        for q in range(NSUB):
            if q - 1 in scs:
                for cp in scs.pop(q - 1):
                    cp.wait()
            if q + 1 < NSUB:
                cps[q + 1] = issue_in(rb, q + 1, (q + 1) % 2)
            for cp in cps.pop(q):
                cp.wait()
            compute(q % 2)
            scs[q] = issue_scatter(q, q % 2)
        for cp in scs.pop(NSUB - 1):
            cp.wait()
        return carry

    lax.fori_loop(0, nchunks, chunk_body, 0)

    @pl.when(sid == 15)
    def _tail():
        rb = NCHUNKS * CHUNK_ROWS
        load_idx(rb, TAIL_ROWS)
        for q in range(TAIL_ROWS // SUB_ROWS):
            for cp in issue_in(rb, q, q % 2):
                cp.wait()
            compute(q % 2)
            for cp in issue_scatter(q, q % 2):
                cp.wait()

    plsc.subcore_barrier()
    g0 = 781 * sid + jnp.minimum(sid, 4)
    pltpu.sync_copy(aggsp.at[pl.ds(g0 * 8, 6248)],
                    out_hbm.at[pl.ds(cid * N + g0 * 8, 6248)])

    @pl.when(sid < 4)
    def _wb_extra():
        off = (g0 + 781) * 8
        pltpu.sync_copy(aggsp.at[pl.ds(off, 8)],
                        out_hbm.at[pl.ds(cid * N + off, 8)])


def _sc_agg(h_cat, e_packed, src2, dst2):
    h_cat = h_cat.reshape(2, N, HH)
    mesh = plsc.VectorSubcoreMesh(core_axis_name="c", subcore_axis_name="s",
                                  num_cores=2, num_subcores=16)
    return pl.kernel(
        _sc_agg_body,
        out_type=jax.ShapeDtypeStruct((2 * N, HH), jnp.float32),
        mesh=mesh,
        scratch_types=[
            pltpu.VMEM((CHUNK_ROWS, 128), jnp.int32),
            pltpu.VMEM((CHUNK_ROWS, 128), jnp.int32),
            pltpu.VMEM((SUB_E, HH), jnp.float32),
            pltpu.VMEM((SUB_E, HH), jnp.float32),
            pltpu.VMEM((SUB_P, 128), jnp.float32),
            pltpu.VMEM((SUB_P, 128), jnp.float32),
            pltpu.VMEM_SHARED((SP_ROWS, HH), jnp.float32),
            pltpu.SemaphoreType.DMA,
            pltpu.SemaphoreType.DMA,
            pltpu.SemaphoreType.DMA,
            pltpu.SemaphoreType.DMA,
            pltpu.SemaphoreType.DMA,
            pltpu.SemaphoreType.DMA,
        ],
        compiler_params=pltpu.CompilerParams(use_tc_tiling_on_sc=False),
    )(h_cat, e_packed, src2, dst2)


def kernel(x, edge_index, edge_attr, type_embed, feat_W, feat_b, edge_W,
           edge_b, W1_0, b1_0, W2_0, b2_0, eps_0, W1_1, b1_1, W2_1, b2_1,
           eps_1, W1_2, b1_2, W2_2, b2_2, eps_2):
    w_pad = jnp.concatenate([jnp.zeros((1, H), jnp.float32), feat_W], axis=0)

    def col_halves(m):
        return jnp.stack([m[:, :HH], m[:, HH:]])

    h = _node_embed(x, col_halves(type_embed), col_halves(w_pad),
                    col_halves(feat_b.reshape(1, H)).reshape(2, 1, HH))

    ea_packed = edge_attr.reshape(EP, 128)
    eye8 = jnp.eye(8, dtype=jnp.float32)
    wk = jnp.stack([jnp.kron(eye8, edge_W[:, :HH]),
                    jnp.kron(eye8, edge_W[:, HH:])])
    b_tile = jnp.stack([jnp.tile(edge_b[:HH], 8),
                        jnp.tile(edge_b[HH:], 8)]).reshape(2, 1, 128)
    e_packed = _edge_embed(ea_packed, wk, b_tile)

    src2 = edge_index[0].reshape(ROWS, 128)
    dst2 = edge_index[1].reshape(ROWS, 128)

    layers = [
        (W1_0, b1_0, W2_0, b2_0, eps_0, True, True),
        (W1_1, b1_1, W2_1, b2_1, eps_1, True, True),
        (W1_2, b1_2, W2_2, b2_2, eps_2, False, False),
    ]
    for w1, b1, w2, b2, eps, relu_out, split_out in layers:
        agg = _sc_agg(h, e_packed, src2, dst2)
        k1lo = jnp.kron(eye8, w1[:HH, :])
        k1hi = jnp.kron(eye8, w1[HH:, :])
        b1t = jnp.tile(b1, 8).reshape(1, 256)
        if split_out:
            k2 = jnp.stack([jnp.kron(eye8, w2[:, :HH]),
                            jnp.kron(eye8, w2[:, HH:])])
            b2t = jnp.stack([jnp.tile(b2[:HH], 8),
                             jnp.tile(b2[HH:], 8)]).reshape(2, 1, 128)
        else:
            k2 = jnp.kron(eye8, w2)
            b2t = jnp.tile(b2, 8).reshape(1, 128)
        out = _mlp(h.reshape(2, NPB, 128), agg.reshape(2, NPB, 128),
                   k1lo, k1hi, b1t, k2, b2t, eps.reshape(1, 1),
                   relu_out, split_out)
        h = out.reshape(2 * N, HH) if split_out else out.reshape(N, PE)
    return h


# speedup vs baseline: 10.6293x; 1.0811x over previous
import functools

import jax
import jax.numpy as jnp
from jax import lax
from jax.experimental import pallas as pl
from jax.experimental.pallas import tpu as pltpu
from jax.experimental.pallas import tpu_sc as plsc

N = 100000
E = 1600000
NT = 32
NF = 9
NEF = 16
H = 32
PE = 16
HH = H // 2
EP = E // 8

ROWS = E // 128
SP_ROWS = 100096
STRIPE = SP_ROWS // 16
CHUNK_ROWS = 16
SUB_ROWS = 2
SUB_E = SUB_ROWS * 128
SUB_P = SUB_E // 8
NSUB = CHUNK_ROWS // SUB_ROWS
NCHUNKS = ROWS // CHUNK_ROWS
TAIL_ROWS = ROWS - NCHUNKS * CHUNK_ROWS
ZCOPIES = STRIPE // SUB_E
ZREM = STRIPE - ZCOPIES * SUB_E

BN = 2000
NBLK = N // BN
NPB = N // 8
BNP = BN // 8
BEP = 8000
EBLK = EP // BEP


def _node_embed_body(x_ref, te_ref, w_ref, b_ref, o_ref):
    xb = x_ref[...]
    t = xb[:, 0:1]
    iot = lax.broadcasted_iota(jnp.int32, (1, NT), 1).astype(jnp.float32)
    oh = (t == iot).astype(jnp.float32)
    h = jnp.dot(oh, te_ref[0], preferred_element_type=jnp.float32)
    h = h + jnp.dot(xb, w_ref[0], preferred_element_type=jnp.float32)
    o_ref[...] = h + b_ref[0]


def _edge_embed_body(ea_ref, wk_ref, b_ref, o_ref):
    o_ref[...] = (
        jnp.dot(ea_ref[...], wk_ref[0], preferred_element_type=jnp.float32)
        + b_ref[0]
    )


def _mlp_body(hlo_ref, hhi_ref, alo_ref, ahi_ref, k1lo_ref, k1hi_ref,
              b1_ref, k2_ref, b2_ref, eps_ref, o_ref, *, relu_out, split_out):
    s = 1.0 + eps_ref[0, 0]
    zlo = s * hlo_ref[0] + alo_ref[0]
    zhi = s * hhi_ref[0] + ahi_ref[0]
    z = (jnp.dot(zlo, k1lo_ref[...], preferred_element_type=jnp.float32)
         + jnp.dot(zhi, k1hi_ref[...], preferred_element_type=jnp.float32)
         + b1_ref[...])
    z = jnp.maximum(z, 0.0)
    k2 = k2_ref[0] if split_out else k2_ref[...]
    b2 = b2_ref[0] if split_out else b2_ref[...]
    out = jnp.dot(z, k2, preferred_element_type=jnp.float32) + b2
    if relu_out:
        out = jnp.maximum(out, 0.0)
    if split_out:
        o_ref[0] = out
    else:
        o_ref[...] = out


def _node_embed(x, type_embed, w_pad, b):
    return pl.pallas_call(
        _node_embed_body,
        grid=(2, NBLK),
        in_specs=[
            pl.BlockSpec((BN, NF), lambda c, i: (i, 0)),
            pl.BlockSpec((1, NT, HH), lambda c, i: (c, 0, 0)),
            pl.BlockSpec((1, NF, HH), lambda c, i: (c, 0, 0)),
            pl.BlockSpec((1, 1, HH), lambda c, i: (c, 0, 0)),
        ],
        out_specs=pl.BlockSpec((BN, HH), lambda c, i: (c * NBLK + i, 0)),
        out_shape=jax.ShapeDtypeStruct((2 * N, HH), jnp.float32),
    )(x, type_embed, w_pad, b)


def _edge_embed(ea_packed, wk, b_tile):
    return pl.pallas_call(
        _edge_embed_body,
        grid=(2, EBLK),
        in_specs=[
            pl.BlockSpec((BEP, 128), lambda c, i: (i, 0)),
            pl.BlockSpec((1, 128, 128), lambda c, i: (c, 0, 0)),
            pl.BlockSpec((1, 1, 128), lambda c, i: (c, 0, 0)),
        ],
        out_specs=pl.BlockSpec((BEP, 128), lambda c, i: (c * EBLK + i, 0)),
        out_shape=jax.ShapeDtypeStruct((2 * EP, 128), jnp.float32),
    )(ea_packed, wk, b_tile)


def _mlp(hp, aggp, k1lo, k1hi, b1t, k2, b2t, eps, relu_out, split_out):
    if split_out:
        grid = (2,)
        k2_spec = pl.BlockSpec((1, 256, 128), lambda c: (c, 0, 0))
        b2_spec = pl.BlockSpec((1, 1, 128), lambda c: (c, 0, 0))
        out_specs = pl.BlockSpec((1, NPB, 128), lambda c: (c, 0, 0))
        out_shape = jax.ShapeDtypeStruct((2, NPB, 128), jnp.float32)
    else:
        grid = (1,)
        k2_spec = pl.BlockSpec((256, 128), lambda c: (0, 0))
        b2_spec = pl.BlockSpec((1, 128), lambda c: (0, 0))
        out_specs = pl.BlockSpec((NPB, 128), lambda c: (0, 0))
        out_shape = jax.ShapeDtypeStruct((NPB, 128), jnp.float32)
    lo = lambda c: (0, 0, 0)
    hi = lambda c: (1, 0, 0)
    fixed = lambda c: (0, 0)
    return pl.pallas_call(
        functools.partial(_mlp_body, relu_out=relu_out, split_out=split_out),
        grid=grid,
        in_specs=[
            pl.BlockSpec((1, NPB, 128), lo),
            pl.BlockSpec((1, NPB, 128), hi),
            pl.BlockSpec((1, NPB, 128), lo),
            pl.BlockSpec((1, NPB, 128), hi),
            pl.BlockSpec((128, 256), fixed),
            pl.BlockSpec((128, 256), fixed),
            pl.BlockSpec((1, 256), fixed),
            k2_spec,
            b2_spec,
            pl.BlockSpec((1, 1), fixed),
        ],
        out_specs=out_specs,
        out_shape=out_shape,
    )(hp, hp, aggp, aggp, k1lo, k1hi, b1t, k2, b2t, eps)


def _sc_agg_body(h_hbm, e_hbm, ei_hbm, out_hbm,
                 sbuf, dbuf, g0b, g1b, g2b, e0b, e1b, e2b, aggsp,
                 gsem0, gsem1, gsem2, esem0, esem1, esem2,
                 ssem0, ssem1, ssem2):
    cid = lax.axis_index("c")
    sid = lax.axis_index("s")

    gb = [g0b, g1b, g2b]
    eb = [e0b, e1b, e2b]
    gsem = [gsem0, gsem1, gsem2]
    esem = [esem0, esem1, esem2]
    ssem = [ssem0, ssem1, ssem2]
    eoff = cid * EP

    @plsc.parallel_loop(0, SUB_E, unroll=8)
    def _zero(i):
        g0b[i, pl.ds(0, 16)] = jnp.zeros((16,), jnp.float32)

    zcps = [
        pltpu.async_copy(g0b, aggsp.at[pl.ds(sid * STRIPE + k * SUB_E, SUB_E)],
                         gsem0)
        for k in range(ZCOPIES)
    ]
    if ZREM:
        zcps.append(
            pltpu.async_copy(g0b.at[pl.ds(0, ZREM)],
                             aggsp.at[pl.ds(sid * STRIPE + ZCOPIES * SUB_E,
                                            ZREM)], gsem0))
    for cp in zcps:
        cp.wait()
    plsc.subcore_barrier()

    nchunks = jnp.where(sid < 13, 49, 48)
    chunk0 = 48 * sid + jnp.minimum(sid, 13)

    def issue_in(rb, q, slot):
        g = [
            pltpu.async_copy(
                h_hbm.at[cid].at[sbuf.at[q * SUB_ROWS + j]],
                gb[slot].at[pl.ds(j * 128, 128)], gsem[slot])
            for j in range(SUB_ROWS)
        ]
        e = pltpu.async_copy(
            e_hbm.at[pl.ds(eoff + (rb + q * SUB_ROWS) * 16, SUB_P)],
            eb[slot], esem[slot])
        return g + [e]

    def compute(slot):
        gref, eref = gb[slot], eb[slot]

        @plsc.parallel_loop(0, SUB_P, unroll=2)
        def _relu_add(i):
            for k in range(8):
                v = gref[i * 8 + k, pl.ds(0, 16)] + eref[i, pl.ds(k * 16, 16)]
                gref[i * 8 + k, pl.ds(0, 16)] = jnp.maximum(v, 0.0)

    def issue_scatter(q, slot):
        return [
            pltpu.async_copy(gb[slot].at[pl.ds(j * 128, 128)],
                             aggsp.at[dbuf.at[q * SUB_ROWS + j]], ssem[slot],
                             add=True)
            for j in range(SUB_ROWS)
        ]

    def load_idx(rb, nrows):
        pltpu.sync_copy(ei_hbm.at[0].at[pl.ds(rb, nrows)],
                        sbuf.at[pl.ds(0, nrows)])
        pltpu.sync_copy(ei_hbm.at[1].at[pl.ds(rb, nrows)],
                        dbuf.at[pl.ds(0, nrows)])

    def chunk_body(c, carry):
        rb = (chunk0 + c) * CHUNK_ROWS
        load_idx(rb, CHUNK_ROWS)
        cps = {0: issue_in(rb, 0, 0), 1: issue_in(rb, 1, 1)}
        scs = {}
        for q in range(NSUB):
            if q - 2 in scs:
                for cp in scs.pop(q - 2):
                    cp.wait()
            if q + 1 < NSUB:
                cps[q + 1] = issue_in(rb, q + 1, (q + 1) % 3)
            for cp in cps.pop(q):
                cp.wait()
            compute(q % 3)
            scs[q] = issue_scatter(q, q % 3)
        for q in sorted(scs):
            for cp in scs.pop(q):
                cp.wait()
        return carry

    lax.fori_loop(0, nchunks, chunk_body, 0)

    @pl.when(sid == 15)
    def _tail():
        rb = NCHUNKS * CHUNK_ROWS
        load_idx(rb, TAIL_ROWS)
        for q in range(TAIL_ROWS // SUB_ROWS):
            for cp in issue_in(rb, q, q % 3):
                cp.wait()
            compute(q % 3)
            for cp in issue_scatter(q, q % 3):
                cp.wait()

    plsc.subcore_barrier()
    g0 = 781 * sid + jnp.minimum(sid, 4)
    pltpu.sync_copy(aggsp.at[pl.ds(g0 * 8, 6248)],
                    out_hbm.at[pl.ds(cid * N + g0 * 8, 6248)])

    @pl.when(sid < 4)
    def _wb_extra():
        off = (g0 + 781) * 8
        pltpu.sync_copy(aggsp.at[pl.ds(off, 8)],
                        out_hbm.at[pl.ds(cid * N + off, 8)])


def _sc_agg(h_cat, e_packed, ei3):
    h_cat = h_cat.reshape(2, N, HH)
    mesh = plsc.VectorSubcoreMesh(core_axis_name="c", subcore_axis_name="s",
                                  num_cores=2, num_subcores=16)
    return pl.kernel(
        _sc_agg_body,
        out_type=jax.ShapeDtypeStruct((2 * N, HH), jnp.float32),
        mesh=mesh,
        scratch_types=[
            pltpu.VMEM((CHUNK_ROWS, 128), jnp.int32),
            pltpu.VMEM((CHUNK_ROWS, 128), jnp.int32),
            pltpu.VMEM((SUB_E, HH), jnp.float32),
            pltpu.VMEM((SUB_E, HH), jnp.float32),
            pltpu.VMEM((SUB_E, HH), jnp.float32),
            pltpu.VMEM((SUB_P, 128), jnp.float32),
            pltpu.VMEM((SUB_P, 128), jnp.float32),
            pltpu.VMEM((SUB_P, 128), jnp.float32),
            pltpu.VMEM_SHARED((SP_ROWS, HH), jnp.float32),
            pltpu.SemaphoreType.DMA,
            pltpu.SemaphoreType.DMA,
            pltpu.SemaphoreType.DMA,
            pltpu.SemaphoreType.DMA,
            pltpu.SemaphoreType.DMA,
            pltpu.SemaphoreType.DMA,
            pltpu.SemaphoreType.DMA,
            pltpu.SemaphoreType.DMA,
            pltpu.SemaphoreType.DMA,
        ],
        compiler_params=pltpu.CompilerParams(use_tc_tiling_on_sc=False),
    )(h_cat, e_packed, ei3)


def kernel(x, edge_index, edge_attr, type_embed, feat_W, feat_b, edge_W,
           edge_b, W1_0, b1_0, W2_0, b2_0, eps_0, W1_1, b1_1, W2_1, b2_1,
           eps_1, W1_2, b1_2, W2_2, b2_2, eps_2):
    w_pad = jnp.concatenate([jnp.zeros((1, H), jnp.float32), feat_W], axis=0)

    def col_halves(m):
        return jnp.stack([m[:, :HH], m[:, HH:]])

    h = _node_embed(x, col_halves(type_embed), col_halves(w_pad),
                    col_halves(feat_b.reshape(1, H)).reshape(2, 1, HH))

    ea_packed = edge_attr.reshape(EP, 128)
    eye8 = jnp.eye(8, dtype=jnp.float32)
    wk = jnp.stack([jnp.kron(eye8, edge_W[:, :HH]),
                    jnp.kron(eye8, edge_W[:, HH:])])
    b_tile = jnp.stack([jnp.tile(edge_b[:HH], 8),
                        jnp.tile(edge_b[HH:], 8)]).reshape(2, 1, 128)
    e_packed = _edge_embed(ea_packed, wk, b_tile)

    ei3 = edge_index.reshape(2, ROWS, 128)

    layers = [
        (W1_0, b1_0, W2_0, b2_0, eps_0, True, True),
        (W1_1, b1_1, W2_1, b2_1, eps_1, True, True),
        (W1_2, b1_2, W2_2, b2_2, eps_2, False, False),
    ]
    for w1, b1, w2, b2, eps, relu_out, split_out in layers:
        agg = _sc_agg(h, e_packed, ei3)
        k1lo = jnp.kron(eye8, w1[:HH, :])
        k1hi = jnp.kron(eye8, w1[HH:, :])
        b1t = jnp.tile(b1, 8).reshape(1, 256)
        if split_out:
            k2 = jnp.stack([jnp.kron(eye8, w2[:, :HH]),
                            jnp.kron(eye8, w2[:, HH:])])
            b2t = jnp.stack([jnp.tile(b2[:HH], 8),
                             jnp.tile(b2[HH:], 8)]).reshape(2, 1, 128)
        else:
            k2 = jnp.kron(eye8, w2)
            b2t = jnp.tile(b2, 8).reshape(1, 128)
        out = _mlp(h.reshape(2, NPB, 128), agg.reshape(2, NPB, 128),
                   k1lo, k1hi, b1t, k2, b2t, eps.reshape(1, 1),
                   relu_out, split_out)
        h = out.reshape(2 * N, HH) if split_out else out.reshape(N, PE)
    return h
